# baseline (speedup 1.0000x reference)
import functools

import jax
import jax.numpy as jnp
from jax import lax
from jax.experimental import pallas as pl
from jax.experimental.pallas import tpu as pltpu
from jax.experimental.pallas import tpu_sc as plsc

_NC = 2
_NS = 16
_NW = _NC * _NS
_CHUNK = 256


def _sc_body(nf_chunks, n_full, tail_rows, per_w,
             feat_hbm, mem_hbm, fit_hbm, out_mem_hbm, out_fit_hbm,
             buf0, buf1, fbuf0, fbuf1, ones_v,
             sem_in0, sem_in1, sem_out0, sem_out1):
    wid = lax.axis_index("s") * _NC + lax.axis_index("c")
    c0 = wid * per_w

    for i in range(_CHUNK // 16):
        ones_v[pl.ds(i * 16, 16)] = jnp.ones((16,), jnp.float32)

    bufs = (buf0, buf1)
    fbufs = (fbuf0, fbuf1)
    sin = (sem_in0, sem_in1)
    sout = (sem_out0, sem_out1)

    def start_in(c, slot):
        buf, fbuf = bufs[slot], fbufs[slot]
        row = c * _CHUNK

        @pl.when(c < nf_chunks)
        def _():
            pltpu.async_copy(feat_hbm.at[pl.ds(row, _CHUNK)], buf, sin[slot])

        @pl.when((c >= nf_chunks) & (c < n_full))
        def _():
            pltpu.async_copy(mem_hbm.at[pl.ds(row, _CHUNK)], buf, sin[slot])
            pltpu.async_copy(fit_hbm.at[pl.ds(row, _CHUNK)], fbuf, sin[slot])

        @pl.when(c == n_full)
        def _():
            pltpu.async_copy(mem_hbm.at[pl.ds(row, tail_rows)],
                             buf.at[pl.ds(0, tail_rows)], sin[slot])
            pltpu.async_copy(fit_hbm.at[pl.ds(row, tail_rows)],
                             fbuf.at[pl.ds(0, tail_rows)], sin[slot])

    def wait_in(c, slot):
        buf, fbuf = bufs[slot], fbufs[slot]

        @pl.when(c < nf_chunks)
        def _():
            pltpu.make_async_copy(feat_hbm.at[pl.ds(0, _CHUNK)], buf,
                                  sin[slot]).wait()

        @pl.when((c >= nf_chunks) & (c < n_full))
        def _():
            pltpu.make_async_copy(mem_hbm.at[pl.ds(0, _CHUNK)], buf,
                                  sin[slot]).wait()
            pltpu.make_async_copy(fit_hbm.at[pl.ds(0, _CHUNK)], fbuf,
                                  sin[slot]).wait()

        @pl.when(c == n_full)
        def _():
            pltpu.make_async_copy(mem_hbm.at[pl.ds(0, tail_rows)],
                                  buf.at[pl.ds(0, tail_rows)], sin[slot]).wait()
            pltpu.make_async_copy(fit_hbm.at[pl.ds(0, tail_rows)],
                                  fbuf.at[pl.ds(0, tail_rows)], sin[slot]).wait()

    def start_out(c, slot):
        buf, fbuf = bufs[slot], fbufs[slot]
        row = c * _CHUNK

        @pl.when(c < nf_chunks)
        def _():
            pltpu.async_copy(buf, out_mem_hbm.at[pl.ds(row, _CHUNK)], sout[slot])
            pltpu.async_copy(ones_v, out_fit_hbm.at[pl.ds(row, _CHUNK)], sout[slot])

        @pl.when((c >= nf_chunks) & (c < n_full))
        def _():
            pltpu.async_copy(buf, out_mem_hbm.at[pl.ds(row, _CHUNK)], sout[slot])
            pltpu.async_copy(fbuf, out_fit_hbm.at[pl.ds(row, _CHUNK)], sout[slot])

        @pl.when(c == n_full)
        def _():
            pltpu.async_copy(buf.at[pl.ds(0, tail_rows)],
                             out_mem_hbm.at[pl.ds(row, tail_rows)], sout[slot])
            pltpu.async_copy(fbuf.at[pl.ds(0, tail_rows)],
                             out_fit_hbm.at[pl.ds(row, tail_rows)], sout[slot])

    def wait_out(c, slot):
        buf, fbuf = bufs[slot], fbufs[slot]

        @pl.when(c < nf_chunks)
        def _():
            pltpu.make_async_copy(buf, out_mem_hbm.at[pl.ds(0, _CHUNK)],
                                  sout[slot]).wait()
            pltpu.make_async_copy(ones_v, out_fit_hbm.at[pl.ds(0, _CHUNK)],
                                  sout[slot]).wait()

        @pl.when((c >= nf_chunks) & (c < n_full))
        def _():
            pltpu.make_async_copy(buf, out_mem_hbm.at[pl.ds(0, _CHUNK)],
                                  sout[slot]).wait()
            pltpu.make_async_copy(fbuf, out_fit_hbm.at[pl.ds(0, _CHUNK)],
                                  sout[slot]).wait()

        @pl.when(c == n_full)
        def _():
            pltpu.make_async_copy(buf.at[pl.ds(0, tail_rows)],
                                  out_mem_hbm.at[pl.ds(0, tail_rows)],
                                  sout[slot]).wait()
            pltpu.make_async_copy(fbuf.at[pl.ds(0, tail_rows)],
                                  out_fit_hbm.at[pl.ds(0, tail_rows)],
                                  sout[slot]).wait()

    start_in(c0, 0)
    for k in range(per_w):
        c = c0 + k
        cur = k % 2
        nxt = (k + 1) % 2
        if k >= 1:
            wait_out(c - 1, nxt)
        if k + 1 < per_w:
            start_in(c + 1, nxt)
        wait_in(c, cur)
        start_out(c, cur)
    wait_out(c0 + per_w - 1, (per_w - 1) % 2)


def _sc_call(features, memory, fitness):
    B = features.shape[0]
    cap, dim = memory.shape
    n_chunks = (cap + _CHUNK - 1) // _CHUNK
    n_full = cap // _CHUNK
    tail_rows = cap - n_full * _CHUNK
    nf_chunks = B // _CHUNK
    per_w = (n_chunks + _NW - 1) // _NW

    mesh = plsc.VectorSubcoreMesh(core_axis_name="c", subcore_axis_name="s")
    kfn = pl.kernel(
        functools.partial(_sc_body, nf_chunks, n_full, tail_rows, per_w),
        mesh=mesh,
        out_type=(
            jax.ShapeDtypeStruct((cap, dim), memory.dtype),
            jax.ShapeDtypeStruct((cap,), fitness.dtype),
        ),
        scratch_types=[
            pltpu.VMEM((_CHUNK, dim), jnp.float32),
            pltpu.VMEM((_CHUNK, dim), jnp.float32),
            pltpu.VMEM((_CHUNK,), jnp.float32),
            pltpu.VMEM((_CHUNK,), jnp.float32),
            pltpu.VMEM((_CHUNK,), jnp.float32),
            pltpu.SemaphoreType.DMA,
            pltpu.SemaphoreType.DMA,
            pltpu.SemaphoreType.DMA,
            pltpu.SemaphoreType.DMA,
        ],
    )
    return kfn(features, memory, fitness)


def kernel(features, memory, fitness):
    return _sc_call(features, memory, fitness)


# speedup vs baseline: 1.1724x; 1.1724x over previous
import functools

import jax
import jax.numpy as jnp
from jax import lax
from jax.experimental import pallas as pl
from jax.experimental.pallas import tpu as pltpu
from jax.experimental.pallas import tpu_sc as plsc

_NC = 2
_NS = 16
_NW = _NC * _NS
_FCHUNK = 1024
_BLOCK = 8192


def _sc_fit_body(nf_chunks, n_full, tail_n, per_w,
                 fit_hbm, out_fit_hbm, buf0, buf1, ones_v, sem0, sem1):
    wid = lax.axis_index("s") * _NC + lax.axis_index("c")
    c0 = wid * per_w

    for i in range(_FCHUNK // 16):
        ones_v[pl.ds(i * 16, 16)] = jnp.ones((16,), jnp.float32)

    bufs = (buf0, buf1)
    sems = (sem0, sem1)

    def start_in(c, slot):
        @pl.when((c >= nf_chunks) & (c < n_full))
        def _():
            pltpu.async_copy(fit_hbm.at[pl.ds(c * _FCHUNK, _FCHUNK)],
                             bufs[slot], sems[slot])

        @pl.when(c == n_full)
        def _():
            pltpu.async_copy(fit_hbm.at[pl.ds(c * _FCHUNK, tail_n)],
                             bufs[slot].at[pl.ds(0, tail_n)], sems[slot])

    def wait_in(c, slot):
        @pl.when((c >= nf_chunks) & (c < n_full))
        def _():
            pltpu.make_async_copy(fit_hbm.at[pl.ds(0, _FCHUNK)], bufs[slot],
                                  sems[slot]).wait()

        @pl.when(c == n_full)
        def _():
            pltpu.make_async_copy(fit_hbm.at[pl.ds(0, tail_n)],
                                  bufs[slot].at[pl.ds(0, tail_n)],
                                  sems[slot]).wait()

    def write_out(c, slot):
        @pl.when(c < nf_chunks)
        def _():
            pltpu.sync_copy(ones_v, out_fit_hbm.at[pl.ds(c * _FCHUNK, _FCHUNK)])

        @pl.when((c >= nf_chunks) & (c < n_full))
        def _():
            pltpu.sync_copy(bufs[slot], out_fit_hbm.at[pl.ds(c * _FCHUNK, _FCHUNK)])

        @pl.when(c == n_full)
        def _():
            pltpu.sync_copy(bufs[slot].at[pl.ds(0, tail_n)],
                            out_fit_hbm.at[pl.ds(c * _FCHUNK, tail_n)])

    start_in(c0, 0)
    for k in range(per_w):
        c = c0 + k
        cur = k % 2
        if k + 1 < per_w:
            start_in(c + 1, (k + 1) % 2)
        wait_in(c, cur)
        write_out(c, cur)


def _sc_fitness(fitness, B):
    cap = fitness.shape[0]
    n_chunks = (cap + _FCHUNK - 1) // _FCHUNK
    n_full = cap // _FCHUNK
    tail_n = cap - n_full * _FCHUNK
    nf_chunks = B // _FCHUNK
    per_w = (n_chunks + _NW - 1) // _NW
    mesh = plsc.VectorSubcoreMesh(core_axis_name="c", subcore_axis_name="s")
    kfn = pl.kernel(
        functools.partial(_sc_fit_body, nf_chunks, n_full, tail_n, per_w),
        mesh=mesh,
        out_type=jax.ShapeDtypeStruct((cap,), fitness.dtype),
        scratch_types=[
            pltpu.VMEM((_FCHUNK,), jnp.float32),
            pltpu.VMEM((_FCHUNK,), jnp.float32),
            pltpu.VMEM((_FCHUNK,), jnp.float32),
            pltpu.SemaphoreType.DMA,
            pltpu.SemaphoreType.DMA,
        ],
    )
    return kfn(fitness)


def _tc_mem_body(nf, feat_ref, mem_ref, out_mem_ref):
    i = pl.program_id(0)

    @pl.when(i < nf)
    def _():
        out_mem_ref[...] = feat_ref[...]

    @pl.when(i >= nf)
    def _():
        out_mem_ref[...] = mem_ref[...]


def _tc_memory(features, memory):
    B = features.shape[0]
    cap, dim = memory.shape
    block = _BLOCK if B % _BLOCK == 0 else 2048
    nf = B // block
    grid = (cap + block - 1) // block
    return pl.pallas_call(
        functools.partial(_tc_mem_body, nf),
        grid=(grid,),
        out_shape=jax.ShapeDtypeStruct((cap, dim), memory.dtype),
        in_specs=[
            pl.BlockSpec((block, dim), lambda i: (jnp.minimum(i, nf - 1), 0)),
            pl.BlockSpec((block, dim), lambda i: (jnp.maximum(i, nf), 0)),
        ],
        out_specs=pl.BlockSpec((block, dim), lambda i: (i, 0)),
    )(features, memory)


def kernel(features, memory, fitness):
    B = features.shape[0]
    out_fit = _sc_fitness(fitness, B)
    out_mem = _tc_memory(features, memory)
    return out_mem, out_fit


# speedup vs baseline: 1.1746x; 1.0019x over previous
import functools

import jax
import jax.numpy as jnp
from jax import lax
from jax.experimental import pallas as pl
from jax.experimental.pallas import tpu as pltpu
from jax.experimental.pallas import tpu_sc as plsc

_NC = 2
_NS = 16
_NW = _NC * _NS
_FCHUNK = 1024
_BLOCK = 8192


def _sc_fit_body(nf_chunks, n_full, tail_n, per_w,
                 fit_hbm, out_fit_hbm, buf0, buf1, ones_v, sem0, sem1):
    wid = lax.axis_index("s") * _NC + lax.axis_index("c")
    c0 = wid * per_w

    for i in range(_FCHUNK // 16):
        ones_v[pl.ds(i * 16, 16)] = jnp.ones((16,), jnp.float32)

    bufs = (buf0, buf1)
    sems = (sem0, sem1)

    def start_in(c, slot):
        @pl.when((c >= nf_chunks) & (c < n_full))
        def _():
            pltpu.async_copy(fit_hbm.at[pl.ds(c * _FCHUNK, _FCHUNK)],
                             bufs[slot], sems[slot])

        @pl.when(c == n_full)
        def _():
            pltpu.async_copy(fit_hbm.at[pl.ds(c * _FCHUNK, tail_n)],
                             bufs[slot].at[pl.ds(0, tail_n)], sems[slot])

    def wait_in(c, slot):
        @pl.when((c >= nf_chunks) & (c < n_full))
        def _():
            pltpu.make_async_copy(fit_hbm.at[pl.ds(0, _FCHUNK)], bufs[slot],
                                  sems[slot]).wait()

        @pl.when(c == n_full)
        def _():
            pltpu.make_async_copy(fit_hbm.at[pl.ds(0, tail_n)],
                                  bufs[slot].at[pl.ds(0, tail_n)],
                                  sems[slot]).wait()

    def write_out(c, slot):
        @pl.when(c < nf_chunks)
        def _():
            pltpu.sync_copy(ones_v, out_fit_hbm.at[pl.ds(c * _FCHUNK, _FCHUNK)])

        @pl.when((c >= nf_chunks) & (c < n_full))
        def _():
            pltpu.sync_copy(bufs[slot], out_fit_hbm.at[pl.ds(c * _FCHUNK, _FCHUNK)])

        @pl.when(c == n_full)
        def _():
            pltpu.sync_copy(bufs[slot].at[pl.ds(0, tail_n)],
                            out_fit_hbm.at[pl.ds(c * _FCHUNK, tail_n)])

    start_in(c0, 0)
    for k in range(per_w):
        c = c0 + k
        cur = k % 2
        if k + 1 < per_w:
            start_in(c + 1, (k + 1) % 2)
        wait_in(c, cur)
        write_out(c, cur)


def _sc_fitness(fitness, B):
    cap = fitness.shape[0]
    n_chunks = (cap + _FCHUNK - 1) // _FCHUNK
    n_full = cap // _FCHUNK
    tail_n = cap - n_full * _FCHUNK
    nf_chunks = B // _FCHUNK
    per_w = (n_chunks + _NW - 1) // _NW
    mesh = plsc.VectorSubcoreMesh(core_axis_name="c", subcore_axis_name="s")
    kfn = pl.kernel(
        functools.partial(_sc_fit_body, nf_chunks, n_full, tail_n, per_w),
        mesh=mesh,
        out_type=jax.ShapeDtypeStruct((cap,), fitness.dtype),
        scratch_types=[
            pltpu.VMEM((_FCHUNK,), jnp.float32),
            pltpu.VMEM((_FCHUNK,), jnp.float32),
            pltpu.VMEM((_FCHUNK,), jnp.float32),
            pltpu.SemaphoreType.DMA,
            pltpu.SemaphoreType.DMA,
        ],
    )
    return kfn(fitness)


def _tc_mem_body(nf, feat_ref, mem_ref, out_mem_ref):
    i = pl.program_id(0)

    @pl.when(i < nf)
    def _():
        out_mem_ref[...] = feat_ref[...]

    @pl.when(i >= nf)
    def _():
        out_mem_ref[...] = mem_ref[...]


def _tc_memory(features, memory):
    B = features.shape[0]
    cap, dim = memory.shape
    block = _BLOCK if B % _BLOCK == 0 else 2048
    nf = B // block
    grid = (cap + block - 1) // block
    return pl.pallas_call(
        functools.partial(_tc_mem_body, nf),
        grid=(grid,),
        out_shape=jax.ShapeDtypeStruct((cap, dim), memory.dtype),
        in_specs=[
            pl.BlockSpec((block, dim), lambda i: (jnp.minimum(i, nf - 1), 0)),
            pl.BlockSpec((block, dim), lambda i: (jnp.maximum(i, nf), 0)),
        ],
        out_specs=pl.BlockSpec((block, dim), lambda i: (i, 0)),
    )(features, memory)


def kernel(features, memory, fitness):
    B = features.shape[0]
    out_mem = _tc_memory(features, memory)
    out_fit = _sc_fitness(fitness, B)
    return out_mem, out_fit


# speedup vs baseline: 1.7207x; 1.4650x over previous
import jax
import jax.numpy as jnp
from jax.experimental import pallas as pl
from jax.experimental.pallas import tpu as pltpu

_BLOCK = 8192


def _emb_write_body(nf, feat_ref, mem_ref, fit_ref, out_mem_ref, out_fit_ref):
    i = pl.program_id(0)

    @pl.when(i < nf)
    def _():
        out_mem_ref[...] = feat_ref[...]
        out_fit_ref[...] = jnp.ones_like(out_fit_ref)

    @pl.when(i >= nf)
    def _():
        out_mem_ref[...] = mem_ref[...]
        out_fit_ref[...] = fit_ref[...]


def kernel(features, memory, fitness):
    B = features.shape[0]
    cap, dim = memory.shape
    block = _BLOCK if B % _BLOCK == 0 else 2048
    nf = B // block
    grid = (cap + block - 1) // block

    def feat_map(i):
        return (jnp.minimum(i, nf - 1), 0)

    def mem_map(i):
        return (jnp.maximum(i, nf), 0)

    def fit_map(i):
        return (jnp.maximum(i, nf),)

    import functools
    out_mem, out_fit = pl.pallas_call(
        functools.partial(_emb_write_body, nf),
        grid=(grid,),
        out_shape=(
            jax.ShapeDtypeStruct((cap, dim), memory.dtype),
            jax.ShapeDtypeStruct((cap,), fitness.dtype),
        ),
        in_specs=[
            pl.BlockSpec((block, dim), feat_map),
            pl.BlockSpec((block, dim), mem_map),
            pl.BlockSpec((block,), fit_map),
        ],
        out_specs=(
            pl.BlockSpec((block, dim), lambda i: (i, 0)),
            pl.BlockSpec((block,), lambda i: (i,)),
        ),
    )(features, memory, fitness)
    return out_mem, out_fit
